# baseline (device time: 23680 ns/iter reference)
import jax
import jax.numpy as jnp
from jax import lax
from jax.experimental import pallas as pl
from jax.experimental.pallas import tpu as pltpu

HALF = 512
K = 8
CH = HALF // K
W = 2


def kernel(x):
    m_per, n = x.shape

    def body(x_ref, out_ref, sx_sems, rx_sems, sy_sems, ry_sems, copy_sem):
        my_x = lax.axis_index("x")
        my_y = lax.axis_index("y")
        other_x = 1 - my_x
        other_y = 1 - my_y

        barrier_sem = pltpu.get_barrier_semaphore()
        pl.semaphore_signal(
            barrier_sem, inc=1,
            device_id=(other_x, my_y), device_id_type=pl.DeviceIdType.MESH,
        )
        pl.semaphore_signal(
            barrier_sem, inc=1,
            device_id=(my_x, other_y), device_id_type=pl.DeviceIdType.MESH,
        )
        pl.semaphore_wait(barrier_sem, 2)

        src_off = my_y * HALF
        dst_off = my_x * m_per + my_y * HALF
        rdmas_x = []
        for c in range(K):
            r = pltpu.make_async_remote_copy(
                src_ref=x_ref.at[pl.ds(src_off + c * CH, CH)],
                dst_ref=out_ref.at[pl.ds(dst_off + c * CH, CH)],
                send_sem=sx_sems.at[c],
                recv_sem=rx_sems.at[c],
                device_id=(other_x, my_y),
                device_id_type=pl.DeviceIdType.MESH,
            )
            rdmas_x.append(r)
        for c in range(W):
            rdmas_x[c].start()

        local_copy = pltpu.make_async_copy(
            x_ref, out_ref.at[pl.ds(my_x * m_per, m_per)], copy_sem
        )
        local_copy.start()

        fwd_off = other_x * m_per + my_y * HALF
        rdmas_y = []
        for c in range(K):
            rdmas_x[c].wait_recv()
            r = pltpu.make_async_remote_copy(
                src_ref=out_ref.at[pl.ds(fwd_off + c * CH, CH)],
                dst_ref=out_ref.at[pl.ds(fwd_off + c * CH, CH)],
                send_sem=sy_sems.at[c],
                recv_sem=ry_sems.at[c],
                device_id=(my_x, other_y),
                device_id_type=pl.DeviceIdType.MESH,
            )
            r.start()
            rdmas_y.append(r)
            if c + W < K:
                rdmas_x[c + W].start()

        for c in range(K):
            rdmas_y[c].wait_recv()
        local_copy.wait()
        for c in range(K):
            rdmas_x[c].wait_send()
            rdmas_y[c].wait_send()

    return pl.pallas_call(
        body,
        out_shape=jax.ShapeDtypeStruct((2 * m_per, n), x.dtype),
        in_specs=[pl.BlockSpec(memory_space=pltpu.VMEM)],
        out_specs=pl.BlockSpec(memory_space=pltpu.VMEM),
        scratch_shapes=[
            pltpu.SemaphoreType.DMA((K,)),
            pltpu.SemaphoreType.DMA((K,)),
            pltpu.SemaphoreType.DMA((K,)),
            pltpu.SemaphoreType.DMA((K,)),
            pltpu.SemaphoreType.DMA,
        ],
        compiler_params=pltpu.CompilerParams(collective_id=0),
    )(x)
